# baseline (speedup 1.0000x reference)
import functools

import jax
import jax.numpy as jnp
from jax import lax
from jax.experimental import pallas as pl
from jax.experimental.pallas import tpu as pltpu
from jax.experimental.pallas import tpu_sc as plsc


def _sc_call(x_flat, rt_flat, *, S, H, max_len):
    NW = 32
    RW = S // NW
    CJ = 128
    NCH = S // CJ
    CHUNK = CJ * H
    RTB = (CJ + RW) * H

    mesh = plsc.VectorSubcoreMesh(core_axis_name="c", subcore_axis_name="s")

    @functools.partial(
        pl.kernel,
        mesh=mesh,
        out_type=jax.ShapeDtypeStruct((S * S * H,), jnp.float32),
        scratch_types=[
            pltpu.VMEM((CHUNK,), jnp.float32),
            pltpu.VMEM((RTB,), jnp.float32),
            pltpu.VMEM((CHUNK,), jnp.float32),
        ],
    )
    def k(x_hbm, rt_hbm, out_hbm, xbuf, rtbuf, outbuf):
        wid = lax.axis_index("s") * 2 + lax.axis_index("c")
        i0 = wid * RW
        for jc in range(NCH):
            pltpu.sync_copy(x_hbm.at[pl.ds(jc * CHUNK, CHUNK)], xbuf)
            rt_start = (max_len - (i0 + RW - 1) + jc * CJ) * H
            pltpu.sync_copy(rt_hbm.at[pl.ds(rt_start, RTB)], rtbuf)

            def row_body(r, carry):
                off = (RW - 1 - r) * H

                def inner(v, c):
                    b = v * 16
                    outbuf[pl.ds(b, 16)] = (
                        xbuf[pl.ds(b, 16)] + rtbuf[pl.ds(off + b, 16)]
                    )
                    return c

                lax.fori_loop(0, CHUNK // 16, inner, 0)
                out_start = (i0 + r) * (S * H) + jc * CHUNK
                pltpu.sync_copy(outbuf, out_hbm.at[pl.ds(out_start, CHUNK)])
                return carry

            lax.fori_loop(0, RW, row_body, 0)

    return k(x_flat, rt_flat)


def kernel(x, rel_pos_embeddings):
    _, S, H = x.shape
    n_rows = rel_pos_embeddings.shape[0]
    max_len = (n_rows - 1) // 2
    pad = (-n_rows) % 8
    rt = jnp.pad(jnp.flip(rel_pos_embeddings, axis=0), ((0, pad), (0, 0)))
    out = _sc_call(
        x.reshape(S * H), rt.reshape(-1), S=S, H=H, max_len=max_len
    )
    return out.reshape(S, S, H)


# speedup vs baseline: 3.2480x; 3.2480x over previous
import functools

import jax
import jax.numpy as jnp
from jax import lax
from jax.experimental import pallas as pl
from jax.experimental.pallas import tpu as pltpu
from jax.experimental.pallas import tpu_sc as plsc


def _sc_call(x_flat, rt_flat, *, S, H, max_len):
    NW = 32
    RW = S // NW
    CJ = 128
    NCH = S // CJ
    CHUNK = CJ * H
    RTB = (CJ + RW) * H

    mesh = plsc.VectorSubcoreMesh(core_axis_name="c", subcore_axis_name="s")

    @functools.partial(
        pl.kernel,
        mesh=mesh,
        out_type=jax.ShapeDtypeStruct((S * S * H,), jnp.float32),
        scratch_types=[
            pltpu.VMEM((CHUNK,), jnp.float32),
            pltpu.VMEM((RTB,), jnp.float32),
            pltpu.VMEM((CHUNK,), jnp.float32),
        ],
    )
    def k(x_hbm, rt_hbm, out_hbm, xbuf, rtbuf, outbuf):
        wid = lax.axis_index("s") * 2 + lax.axis_index("c")
        i0 = wid * RW
        for jc in range(NCH):
            pltpu.sync_copy(x_hbm.at[pl.ds(jc * CHUNK, CHUNK)], xbuf)
            rt_start = (max_len - (i0 + RW - 1) + jc * CJ) * H
            pltpu.sync_copy(rt_hbm.at[pl.ds(rt_start, RTB)], rtbuf)

            def row_body(r, carry):
                off = (RW - 1 - r) * H

                @plsc.parallel_loop(0, CHUNK, 16, unroll=8)
                def _inner(b):
                    outbuf[pl.ds(b, 16)] = (
                        xbuf[pl.ds(b, 16)] + rtbuf[pl.ds(off + b, 16)]
                    )

                out_start = (i0 + r) * (S * H) + jc * CHUNK
                pltpu.sync_copy(outbuf, out_hbm.at[pl.ds(out_start, CHUNK)])
                return carry

            lax.fori_loop(0, RW, row_body, 0)

    return k(x_flat, rt_flat)


def kernel(x, rel_pos_embeddings):
    _, S, H = x.shape
    n_rows = rel_pos_embeddings.shape[0]
    max_len = (n_rows - 1) // 2
    pad = (-n_rows) % 8
    rt = jnp.pad(jnp.flip(rel_pos_embeddings, axis=0), ((0, pad), (0, 0)))
    out = _sc_call(
        x.reshape(S * H), rt.reshape(-1), S=S, H=H, max_len=max_len
    )
    return out.reshape(S, S, H)


# speedup vs baseline: 4.6570x; 1.4338x over previous
import functools

import jax
import jax.numpy as jnp
from jax import lax
from jax.experimental import pallas as pl
from jax.experimental.pallas import tpu as pltpu
from jax.experimental.pallas import tpu_sc as plsc


def _sc_call(x_flat, rt_flat, *, S, H, max_len):
    NW = 32
    RW = S // NW
    CJ = 128
    NCH = S // CJ
    CHUNK = CJ * H
    RTB = (CJ + RW) * H

    mesh = plsc.VectorSubcoreMesh(core_axis_name="c", subcore_axis_name="s")

    @functools.partial(
        pl.kernel,
        mesh=mesh,
        out_type=jax.ShapeDtypeStruct((S * S * H,), jnp.float32),
        scratch_types=[
            pltpu.VMEM((CHUNK,), jnp.float32),
            pltpu.VMEM((RTB,), jnp.float32),
            pltpu.VMEM((CHUNK,), jnp.float32),
            pltpu.VMEM((CHUNK,), jnp.float32),
            pltpu.SemaphoreType.DMA,
            pltpu.SemaphoreType.DMA,
        ],
    )
    def k(x_hbm, rt_hbm, out_hbm, xbuf, rtbuf, outa, outb, sema, semb):
        wid = lax.axis_index("s") * 2 + lax.axis_index("c")
        i0 = wid * RW

        def compute(buf, r, jc):
            off = (RW - 1 - r) * H

            @plsc.parallel_loop(0, CHUNK, 16, unroll=8)
            def _inner(b):
                buf[pl.ds(b, 16)] = (
                    xbuf[pl.ds(b, 16)] + rtbuf[pl.ds(off + b, 16)]
                )

        def start(buf, sem, r, jc):
            out_start = (i0 + r) * (S * H) + jc * CHUNK
            pltpu.make_async_copy(
                buf, out_hbm.at[pl.ds(out_start, CHUNK)], sem
            ).start()

        def wait(buf, sem):
            pltpu.make_async_copy(
                buf, out_hbm.at[pl.ds(0, CHUNK)], sem
            ).wait()

        for jc in range(NCH):
            pltpu.sync_copy(x_hbm.at[pl.ds(jc * CHUNK, CHUNK)], xbuf)
            rt_start = (max_len - (i0 + RW - 1) + jc * CJ) * H
            pltpu.sync_copy(rt_hbm.at[pl.ds(rt_start, RTB)], rtbuf)

            if jc == 0:
                compute(outa, 0, 0)
                start(outa, sema, 0, 0)
                compute(outb, 1, 0)
                start(outb, semb, 1, 0)
                lo = 1
            else:
                lo = 0

            def row_pair(rp, carry, jc=jc):
                wait(outa, sema)
                compute(outa, 2 * rp, jc)
                start(outa, sema, 2 * rp, jc)
                wait(outb, semb)
                compute(outb, 2 * rp + 1, jc)
                start(outb, semb, 2 * rp + 1, jc)
                return carry

            lax.fori_loop(lo, RW // 2, row_pair, 0)

        wait(outa, sema)
        wait(outb, semb)

    return k(x_flat, rt_flat)


def kernel(x, rel_pos_embeddings):
    _, S, H = x.shape
    n_rows = rel_pos_embeddings.shape[0]
    max_len = (n_rows - 1) // 2
    pad = (-n_rows) % 8
    rt = jnp.pad(jnp.flip(rel_pos_embeddings, axis=0), ((0, pad), (0, 0)))
    out = _sc_call(
        x.reshape(S * H), rt.reshape(-1), S=S, H=H, max_len=max_len
    )
    return out.reshape(S, S, H)
